# baseline (speedup 1.0000x reference)
import jax
import jax.numpy as jnp
from jax import lax
from jax.experimental import pallas as pl
from jax.experimental.pallas import tpu as pltpu
from jax.experimental.pallas import tpu_sc as plsc

N = 4096
K = 12
ED = 128
L = 6
FF = 4 * ED
SIG_LO = 0.05
SIG_HI = 1.0

BN = 256
NB = N // BN
BR = 256
NRB = N // BR


def _dot(a, b, preferred_element_type=None):
    return jax.lax.dot_general(
        a, b, (((1,), (0,)), ((), ())),
        preferred_element_type=preferred_element_type)


def _bf(x):
    return x.astype(jnp.bfloat16).astype(jnp.float32)

def _knn_body(mm_ref, sqr_ref, sqc_ref, d2k_ref, idx_ref):
    d2 = sqr_ref[...] + sqc_ref[...] - 2.0 * mm_ref[...]
    iota = jax.lax.broadcasted_iota(jnp.int32, (BR, N), 1)
    for k in range(K):
        m = jnp.min(d2, axis=1, keepdims=True)
        sel = jnp.where(d2 <= m, iota, N)
        j = jnp.min(sel, axis=1, keepdims=True)
        d2k_ref[:, k:k + 1] = m
        idx_ref[:, k:k + 1] = j
        d2 = jnp.where(iota == j, jnp.float32(jnp.inf), d2)


def _ln(x, s, b):
    m = jnp.mean(x, axis=1, keepdims=True)
    xc = x - m
    v = jnp.mean(xc * xc, axis=1, keepdims=True)
    return xc / jnp.sqrt(v + 1e-5) * s + b


def _attn(hs, s_, b_, wq, wk, wv, wo, wg, h1_ref, nw_ref):
    scale = jnp.float32(ED ** 0.5)
    q = []
    kk = []
    vv = []
    for k in range(K):
        x = _ln(hs[k], s_, b_)
        q.append(_bf(_dot(x, wq, preferred_element_type=jnp.float32)))
        kk.append(_bf(_dot(x, wk, preferred_element_type=jnp.float32)))
        vv.append(_bf(_dot(x, wv, preferred_element_type=jnp.float32)))
    nsum = None
    for k in range(K):
        sc = jnp.concatenate(
            [jnp.sum(q[k] * kk[m], axis=1, keepdims=True) for m in range(K)],
            axis=1) / scale
        mx = jnp.max(sc, axis=1, keepdims=True)
        es = jnp.exp(sc - mx)
        aw = _bf(es / jnp.sum(es, axis=1, keepdims=True))
        o = None
        for m in range(K):
            t = aw[:, m:m + 1] * vv[m]
            o = t if o is None else o + t
        h1 = hs[k] + _dot(o, wo, preferred_element_type=jnp.float32)
        h1_ref[k] = h1
        nsum = h1 if nsum is None else nsum + h1
    node = nsum / jnp.float32(K)
    nw_ref[...] = _dot(node, wg, preferred_element_type=jnp.float32)


def _ffn(h1_ref, gw_ref, s_, b_, w1, b1, w2, b2):
    out = []
    for k in range(K):
        h2 = h1_ref[k] + gw_ref[k]
        y = _ln(h2, s_, b_)
        f = jnp.maximum(_dot(y, w1, preferred_element_type=jnp.float32) + b1, 0.0)
        out.append(h2 + _dot(f, w2, preferred_element_type=jnp.float32) + b2)
    return out


def _ea_body(d2k_ref, sig_ref, s_ref, b_ref, wq_ref, wk_ref, wv_ref, wo_ref,
             wg_ref, h1_ref, nw_ref):
    sig2 = sig_ref[...]
    hs = []
    for k in range(K):
        dist = jnp.sqrt(jnp.maximum(d2k_ref[:, k:k + 1], 0.0))
        hs.append(jnp.exp(-(dist * dist) / sig2))
    _attn(hs, s_ref[...], b_ref[...], wq_ref[...], wk_ref[...], wv_ref[...],
          wo_ref[...], wg_ref[...], h1_ref, nw_ref)


def _ba_body(h1_ref, gw_ref, s2_ref, b2r_ref, w1_ref, b1_ref, w2_ref,
             b2_ref, s1_ref, b1r_ref, wq_ref, wk_ref, wv_ref, wo_ref,
             wg_ref, h1o_ref, nw_ref):
    hs = _ffn(h1_ref, gw_ref, s2_ref[...], b2r_ref[...], w1_ref[...],
              b1_ref[...], w2_ref[...], b2_ref[...])
    _attn(hs, s1_ref[...], b1r_ref[...], wq_ref[...], wk_ref[...],
          wv_ref[...], wo_ref[...], wg_ref[...], h1o_ref, nw_ref)


def _bd_body(h1_ref, gw_ref, s2_ref, b2r_ref, w1_ref, b1_ref, w2_ref,
             b2_ref, wdt_ref, bd_ref, lg_ref):
    hs = _ffn(h1_ref, gw_ref, s2_ref[...], b2r_ref[...], w1_ref[...],
              b1_ref[...], w2_ref[...], b2_ref[...])
    wdt = _bf(wdt_ref[...])
    bd = bd_ref[...]
    for k in range(K):
        lg = jnp.sum(_bf(hs[k]) * wdt, axis=1, keepdims=True) + bd
        lg_ref[:, k:k + 1] = lg


_SC_NC = 2
_SC_NS = 16
_NW = _SC_NC * _SC_NS
_GB = N // _NW


def _gather_body(table_hbm, idxt_hbm, out_hbm, idx_v, rows_v, gsem, ssem0,
                 ssem1):
    wid = lax.axis_index("s") * _SC_NC + lax.axis_index("c")
    base = wid * _GB
    pltpu.sync_copy(idxt_hbm.at[:, pl.ds(base, _GB)], idx_v)
    ssems = [ssem0, ssem1]
    pending = [None, None]
    for k in range(K):
        b = k & 1
        if pending[b] is not None:
            pending[b].wait()
        pltpu.async_copy(table_hbm.at[idx_v.at[k]], rows_v.at[b], gsem).wait()
        cp = pltpu.async_copy(rows_v.at[b], out_hbm.at[k, pl.ds(base, _GB)],
                              ssems[b])
        pending[b] = cp
    pending[0].wait()
    pending[1].wait()


_sc_gather = pl.kernel(
    _gather_body,
    out_type=jax.ShapeDtypeStruct((K, N, ED), jnp.float32),
    mesh=plsc.VectorSubcoreMesh(core_axis_name="c", subcore_axis_name="s"),
    scratch_types=[
        pltpu.VMEM((K, _GB), jnp.int32),
        pltpu.VMEM((2, _GB, ED), jnp.float32),
        pltpu.SemaphoreType.DMA,
        pltpu.SemaphoreType.DMA,
        pltpu.SemaphoreType.DMA,
    ],
)


def _full_spec(shape):
    nd = len(shape)
    return pl.BlockSpec(shape, lambda i, _nd=nd: (0,) * _nd)


def kernel(coords, ln1_s, ln1_b, Wq, Wk, Wv, Wo, Wg, ln2_s, ln2_b, W1, b1,
           W2, b2, Wd, bd):
    f32 = jnp.float32
    sq = jnp.sum(coords * coords, axis=-1)
    mm = coords @ coords.T
    d2k, idx = pl.pallas_call(
        _knn_body,
        grid=(NRB,),
        in_specs=[
            pl.BlockSpec((BR, N), lambda i: (i, 0)),
            pl.BlockSpec((BR, 1), lambda i: (i, 0)),
            _full_spec((1, N)),
        ],
        out_specs=[
            pl.BlockSpec((BR, K), lambda i: (i, 0)),
            pl.BlockSpec((BR, K), lambda i: (i, 0)),
        ],
        out_shape=[
            jax.ShapeDtypeStruct((N, K), f32),
            jax.ShapeDtypeStruct((N, K), jnp.int32),
        ],
    )(mm, sq.reshape(N, 1), sq.reshape(1, N))

    sigmas = jnp.linspace(0.05, 1.0, ED).astype(f32)
    inv2 = (2.0 * sigmas * sigmas).reshape(1, ED)

    hspec = pl.BlockSpec((K, BN, ED), lambda i: (0, i, 0))
    nkspec = pl.BlockSpec((BN, K), lambda i: (i, 0))
    nwspec = pl.BlockSpec((BN, ED), lambda i: (i, 0))
    vec = _full_spec((1, ED))
    mat = _full_spec((ED, ED))
    h_nw_shape = [
        jax.ShapeDtypeStruct((K, N, ED), f32),
        jax.ShapeDtypeStruct((N, ED), f32),
    ]

    a_weight_specs = [vec, vec, mat, mat, mat, mat, mat]
    b_weight_specs = [vec, vec, _full_spec((ED, FF)), _full_spec((1, FF)),
                      _full_spec((FF, ED)), vec]

    ea = pl.pallas_call(
        _ea_body,
        grid=(NB,),
        in_specs=[nkspec, vec] + a_weight_specs,
        out_specs=[hspec, nwspec],
        out_shape=h_nw_shape,
    )
    ba = pl.pallas_call(
        _ba_body,
        grid=(NB,),
        in_specs=[hspec, hspec] + b_weight_specs + a_weight_specs,
        out_specs=[hspec, nwspec],
        out_shape=h_nw_shape,
    )
    bd_call = pl.pallas_call(
        _bd_body,
        grid=(NB,),
        in_specs=[hspec, hspec] + b_weight_specs + [vec, _full_spec((1, 1))],
        out_specs=nkspec,
        out_shape=jax.ShapeDtypeStruct((N, K), f32),
    )

    def aw(l):
        return (ln1_s[l].reshape(1, ED), ln1_b[l].reshape(1, ED),
                Wq[l], Wk[l], Wv[l], Wo[l], Wg[l])

    def bw(l):
        return (ln2_s[l].reshape(1, ED), ln2_b[l].reshape(1, ED),
                W1[l], b1[l].reshape(1, FF), W2[l], b2[l].reshape(1, ED))

    idxt = idx.T
    h, nw = ea(d2k, inv2, *aw(0))
    for l in range(L - 1):
        g = _sc_gather(nw, idxt)
        h, nw = ba(h, g, *bw(l), *aw(l + 1))
    g = _sc_gather(nw, idxt)
    logits = bd_call(h, g, *bw(L - 1), Wd.reshape(1, ED), bd.reshape(1, 1))

    rows = jnp.broadcast_to(jnp.arange(N)[:, None], (N, K))
    dense = jnp.zeros((N, N), f32).at[rows, idx].add(logits)
    dense = dense + dense.T
    return dense
